# baseline (speedup 1.0000x reference)
import functools

import jax
import jax.numpy as jnp
from jax import lax
from jax.experimental import pallas as pl
from jax.experimental.pallas import tpu as pltpu
from jax.experimental.pallas import tpu_sc as plsc

_N = 8192
_D = 768
_H = 256
_K = 32
_NBLK = 4
_BLK = _N // _NBLK
_RPB = _BLK // 128

_HI = lax.Precision.HIGHEST


def _topk_body(c_ref, wq_ref, bq_ref, wk_ref, e_ref, sc_ref, ix_ref,
               buf0, buf1, logits_s, sem0, sem1):
    bufs = [buf0, buf1]
    sems = [sem0, sem1]
    half = _BLK // 2

    def stream_in_start(j):
        base = j * _BLK
        b = bufs[j % 2]
        s = sems[j % 2]
        pltpu.make_async_copy(e_ref.at[pl.ds(base, half)],
                              b.at[pl.ds(0, half)], s).start()
        pltpu.make_async_copy(e_ref.at[pl.ds(base + half, half)],
                              b.at[pl.ds(half, half)], s).start()

    def stream_in_wait(j):
        b = bufs[j % 2]
        s = sems[j % 2]
        pltpu.make_async_copy(e_ref.at[pl.ds(0, half)],
                              b.at[pl.ds(0, half)], s).wait()
        pltpu.make_async_copy(e_ref.at[pl.ds(0, half)],
                              b.at[pl.ds(half, half)], s).wait()

    stream_in_start(0)
    stream_in_start(1)

    q = jnp.dot(c_ref[...], wq_ref[...], preferred_element_type=jnp.float32,
                precision=_HI) + bq_ref[...]
    w = lax.dot_general(q, wk_ref[...], (((1,), (1,)), ((), ())),
                        preferred_element_type=jnp.float32,
                        precision=_HI)

    w3 = w.reshape(1, 1, _D)
    for j in range(_NBLK):
        stream_in_wait(j)
        b3 = bufs[j % 2][...].reshape(_RPB, 128, _D)
        r = jnp.sum(b3 * w3, axis=2) * 0.0625
        logits_s[pl.ds(j * _RPB, _RPB), :] = r
        if j + 2 < _NBLK:
            stream_in_start(j + 2)

    nchunk = (_N // 128) // 8
    l = logits_s[...].reshape(nchunk, 8, 128)
    m = jnp.max(l)
    work = jnp.exp(l - m)
    zinv = 1.0 / jnp.sum(work)

    big = jnp.int32(2 ** 30)
    flat3 = (lax.broadcasted_iota(jnp.int32, (nchunk, 8, 128), 0) * 1024 +
             lax.broadcasted_iota(jnp.int32, (nchunk, 8, 128), 1) * 128 +
             lax.broadcasted_iota(jnp.int32, (nchunk, 8, 128), 2))

    lane32 = lax.broadcasted_iota(jnp.int32, (1, _K), 1)
    sc_v = jnp.zeros((1, _K), jnp.float32)
    ix_v = jnp.zeros((1, _K), jnp.int32)
    for j in range(_K):
        pj = jnp.max(jnp.max(work, axis=0))
        mask = work == pj
        ij = jnp.min(jnp.where(mask, flat3, big))
        sc_v = jnp.where(lane32 == j, pj * zinv, sc_v)
        ix_v = jnp.where(lane32 == j, ij, ix_v)
        work = jnp.where(mask, 0.0, work)
    sc_ref[...] = sc_v
    ix_ref[...] = ix_v


def _gather_wsum_sc(ix_hbm, e_hbm, rows_hbm, idx_v, rows_v, sem):
    cid = lax.axis_index("c")
    sid = lax.axis_index("s")

    @pl.when(jnp.logical_and(cid == 0, sid < 4))
    def _work():
        w = sid
        pltpu.sync_copy(ix_hbm.at[pl.ds(w * 8, 8)], idx_v)
        pltpu.async_copy(e_hbm.at[idx_v], rows_v, sem).wait()
        pltpu.sync_copy(rows_v, rows_hbm.at[pl.ds(w * 8, 8)])


def _proj_body(rows_ref, sc_ref, wv_ref, bv_ref, wo_ref, bo_ref, out_ref):
    s = sc_ref[...]
    u = jnp.dot(s, rows_ref[...], preferred_element_type=jnp.float32,
                precision=_HI)
    s_sum = jnp.sum(s)
    hv = jnp.dot(u, wv_ref[...],
                 preferred_element_type=jnp.float32) + s_sum * bv_ref[...]
    out = jnp.dot(hv, wo_ref[...],
                  preferred_element_type=jnp.float32) + bo_ref[...]
    out_ref[...] = out


def kernel(class_embedding, entity_embeddings, Wq, bq, Wk, bk, Wv, bv, Wo, bo):
    del bk
    c2 = class_embedding.reshape(1, _D)
    vm = pl.BlockSpec(memory_space=pltpu.VMEM)

    scores, idx = pl.pallas_call(
        _topk_body,
        in_specs=[vm, vm, vm, vm, pl.BlockSpec(memory_space=pl.ANY)],
        out_specs=(vm, vm),
        out_shape=(jax.ShapeDtypeStruct((1, _K), jnp.float32),
                   jax.ShapeDtypeStruct((1, _K), jnp.int32)),
        scratch_shapes=[
            pltpu.VMEM((_BLK, _D), jnp.float32),
            pltpu.VMEM((_BLK, _D), jnp.float32),
            pltpu.VMEM((_N // 128, 128), jnp.float32),
            pltpu.SemaphoreType.DMA,
            pltpu.SemaphoreType.DMA,
        ],
    )(c2, Wq, bq.reshape(1, _H), Wk, entity_embeddings)

    sc_kernel = pl.kernel(
        _gather_wsum_sc,
        out_type=jax.ShapeDtypeStruct((_K, _D), jnp.float32),
        mesh=plsc.VectorSubcoreMesh(core_axis_name="c", subcore_axis_name="s"),
        scratch_types=[
            pltpu.VMEM((8,), jnp.int32),
            pltpu.VMEM((8, _D), jnp.float32),
            pltpu.SemaphoreType.DMA,
        ],
    )
    rows = sc_kernel(idx.reshape(_K), entity_embeddings)

    out = pl.pallas_call(
        _proj_body,
        in_specs=[vm, vm, vm, vm, vm, vm],
        out_specs=vm,
        out_shape=jax.ShapeDtypeStruct((1, _D), jnp.float32),
    )(rows, scores, Wv, bv.reshape(1, _H), Wo, bo.reshape(1, _D))

    return out.reshape(_D)


# speedup vs baseline: 1.7347x; 1.7347x over previous
import jax
import jax.numpy as jnp
from jax import lax
from jax.experimental import pallas as pl
from jax.experimental.pallas import tpu as pltpu

_N = 8192
_D = 768
_H = 256
_K = 32
_NBLK = 4
_BLK = _N // _NBLK
_ROWS_PER_BLK = _BLK // 128

_HI = lax.Precision.HIGHEST


def _fused_body(c_ref, wq_ref, bq_ref, wk_ref, wv_ref, bv_ref, wo_ref, bo_ref,
                e_ref, out_ref, buf0, buf1, logits_s, rows_ref,
                sem0, sem1, semg):
    bufs = [buf0, buf1]
    sems = [sem0, sem1]
    half = _BLK // 2

    def stream_in_start(j):
        base = j * _BLK
        b = bufs[j % 2]
        s = sems[j % 2]
        pltpu.make_async_copy(e_ref.at[pl.ds(base, half)],
                              b.at[pl.ds(0, half)], s).start()
        pltpu.make_async_copy(e_ref.at[pl.ds(base + half, half)],
                              b.at[pl.ds(half, half)], s).start()

    def stream_in_wait(j):
        b = bufs[j % 2]
        s = sems[j % 2]
        pltpu.make_async_copy(e_ref.at[pl.ds(0, half)],
                              b.at[pl.ds(0, half)], s).wait()
        pltpu.make_async_copy(e_ref.at[pl.ds(0, half)],
                              b.at[pl.ds(half, half)], s).wait()

    stream_in_start(0)
    stream_in_start(1)

    q = jnp.dot(c_ref[...], wq_ref[...], preferred_element_type=jnp.float32,
                precision=_HI) + bq_ref[...]
    w = lax.dot_general(q, wk_ref[...], (((1,), (1,)), ((), ())),
                        preferred_element_type=jnp.float32,
                        precision=_HI)

    w3 = w.reshape(1, 1, _D)
    for j in range(_NBLK):
        stream_in_wait(j)
        b3 = bufs[j % 2][...].reshape(_ROWS_PER_BLK, 128, _D)
        r = jnp.sum(b3 * w3, axis=2) * 0.0625
        logits_s[pl.ds(j * _ROWS_PER_BLK, _ROWS_PER_BLK), :] = r
        if j + 2 < _NBLK:
            stream_in_start(j + 2)

    nchunk = (_N // 128) // 8
    l = logits_s[...].reshape(nchunk, 8, 128)
    m = jnp.max(l)
    work = jnp.exp(l - m)
    zinv = 1.0 / jnp.sum(work)

    big = jnp.int32(2 ** 30)
    flat3 = (lax.broadcasted_iota(jnp.int32, (nchunk, 8, 128), 0) * 1024 +
             lax.broadcasted_iota(jnp.int32, (nchunk, 8, 128), 1) * 128 +
             lax.broadcasted_iota(jnp.int32, (nchunk, 8, 128), 2))

    lane32 = lax.broadcasted_iota(jnp.int32, (1, _K), 1)
    sc_v = jnp.zeros((1, _K), jnp.float32)
    for j in range(_K):
        pj = jnp.max(jnp.max(work, axis=0))
        mask = work == pj
        ij = jnp.min(jnp.where(mask, flat3, big))
        sc_v = jnp.where(lane32 == j, pj, sc_v)
        work = jnp.where(mask, 0.0, work)
        pltpu.make_async_copy(e_ref.at[pl.ds(ij, 1)],
                              rows_ref.at[pl.ds(j, 1)], semg).start()

    for j in range(_K):
        pltpu.make_async_copy(e_ref.at[pl.ds(0, 1)],
                              rows_ref.at[pl.ds(j, 1)], semg).wait()

    sc_v = sc_v * zinv
    u = jnp.dot(sc_v, rows_ref[...], preferred_element_type=jnp.float32,
                precision=_HI)
    s_sum = jnp.sum(sc_v)

    hv = jnp.dot(u, wv_ref[...],
                 preferred_element_type=jnp.float32) + s_sum * bv_ref[...]
    out = jnp.dot(hv, wo_ref[...],
                  preferred_element_type=jnp.float32) + bo_ref[...]
    out_ref[...] = out


def kernel(class_embedding, entity_embeddings, Wq, bq, Wk, bk, Wv, bv, Wo, bo):
    del bk
    c2 = class_embedding.reshape(1, _D)

    vm = pl.BlockSpec(memory_space=pltpu.VMEM)
    out = pl.pallas_call(
        _fused_body,
        in_specs=[vm, vm, vm, vm, vm, vm, vm, vm,
                  pl.BlockSpec(memory_space=pl.ANY)],
        out_specs=vm,
        out_shape=jax.ShapeDtypeStruct((1, _D), jnp.float32),
        scratch_shapes=[
            pltpu.VMEM((_BLK, _D), jnp.float32),
            pltpu.VMEM((_BLK, _D), jnp.float32),
            pltpu.VMEM((_N // 128, 128), jnp.float32),
            pltpu.VMEM((_K, _D), jnp.float32),
            pltpu.SemaphoreType.DMA,
            pltpu.SemaphoreType.DMA,
            pltpu.SemaphoreType.DMA,
        ],
    )(c2, Wq, bq.reshape(1, _H), Wk, Wv, bv.reshape(1, _H), Wo,
      bo.reshape(1, _D), entity_embeddings)

    return out.reshape(_D)


# speedup vs baseline: 1.8054x; 1.0407x over previous
import jax
import jax.numpy as jnp
from jax import lax
from jax.experimental import pallas as pl
from jax.experimental.pallas import tpu as pltpu

_N = 8192
_D = 768
_H = 256
_K = 32
_NBLK = 8
_BLK = _N // _NBLK
_ROWS_PER_BLK = _BLK // 128

_HI = lax.Precision.HIGHEST


def _fused_body(c_ref, wq_ref, bq_ref, wk_ref, wv_ref, bv_ref, wo_ref, bo_ref,
                e_ref, out_ref, buf0, buf1, buf2, buf3, logits_s, rows_ref,
                sem0, sem1, sem2, sem3, semg):
    bufs = [buf0, buf1, buf2, buf3]
    sems = [sem0, sem1, sem2, sem3]
    half = _BLK // 2

    def stream_in_start(j):
        base = j * _BLK
        b = bufs[j % 4]
        s = sems[j % 4]
        pltpu.make_async_copy(e_ref.at[pl.ds(base, half)],
                              b.at[pl.ds(0, half)], s).start()
        pltpu.make_async_copy(e_ref.at[pl.ds(base + half, half)],
                              b.at[pl.ds(half, half)], s).start()

    def stream_in_wait(j):
        b = bufs[j % 4]
        s = sems[j % 4]
        pltpu.make_async_copy(e_ref.at[pl.ds(0, half)],
                              b.at[pl.ds(0, half)], s).wait()
        pltpu.make_async_copy(e_ref.at[pl.ds(0, half)],
                              b.at[pl.ds(half, half)], s).wait()

    for _pj in range(4):
        stream_in_start(_pj)

    q = jnp.dot(c_ref[...], wq_ref[...], preferred_element_type=jnp.float32,
                precision=_HI) + bq_ref[...]
    w = lax.dot_general(q, wk_ref[...], (((1,), (1,)), ((), ())),
                        preferred_element_type=jnp.float32,
                        precision=_HI)

    w3 = w.reshape(1, 1, _D)
    for j in range(_NBLK):
        stream_in_wait(j)
        b3 = bufs[j % 4][...].reshape(_ROWS_PER_BLK, 128, _D)
        r = jnp.sum(b3 * w3, axis=2) * 0.0625
        logits_s[pl.ds(j * _ROWS_PER_BLK, _ROWS_PER_BLK), :] = r
        if j + 4 < _NBLK:
            stream_in_start(j + 4)

    nchunk = (_N // 128) // 8
    l = logits_s[...].reshape(nchunk, 8, 128)
    m = jnp.max(l)
    zinv = 1.0 / jnp.sum(jnp.exp(l - m))
    work = l
    neg = jnp.float32(-3e38)

    big = jnp.int32(2 ** 30)
    flat3 = (lax.broadcasted_iota(jnp.int32, (nchunk, 8, 128), 0) * 1024 +
             lax.broadcasted_iota(jnp.int32, (nchunk, 8, 128), 1) * 128 +
             lax.broadcasted_iota(jnp.int32, (nchunk, 8, 128), 2))

    lane32 = lax.broadcasted_iota(jnp.int32, (1, _K), 1)
    sc_v = jnp.zeros((1, _K), jnp.float32)
    idxs = []
    for j in range(_K):
        pj = jnp.max(jnp.max(work, axis=0))
        mask = work == pj
        ij = jnp.min(jnp.where(mask, flat3, big))
        sc_v = jnp.where(lane32 == j, pj, sc_v)
        work = jnp.where(mask, neg, work)
        idxs.append(ij)

    for j in range(_K):
        pltpu.make_async_copy(e_ref.at[pl.ds(idxs[j], 1)],
                              rows_ref.at[pl.ds(j, 1)], semg).start()
    for j in range(_K):
        pltpu.make_async_copy(e_ref.at[pl.ds(0, 1)],
                              rows_ref.at[pl.ds(j, 1)], semg).wait()

    sc_v = jnp.exp(sc_v - m) * zinv
    u = jnp.dot(sc_v, rows_ref[...], preferred_element_type=jnp.float32,
                precision=_HI)
    s_sum = jnp.sum(sc_v)

    hv = jnp.dot(u, wv_ref[...],
                 preferred_element_type=jnp.float32) + s_sum * bv_ref[...]
    out = jnp.dot(hv, wo_ref[...],
                  preferred_element_type=jnp.float32) + bo_ref[...]
    out_ref[...] = out


def kernel(class_embedding, entity_embeddings, Wq, bq, Wk, bk, Wv, bv, Wo, bo):
    del bk
    c2 = class_embedding.reshape(1, _D)

    vm = pl.BlockSpec(memory_space=pltpu.VMEM)
    out = pl.pallas_call(
        _fused_body,
        in_specs=[vm, vm, vm, vm, vm, vm, vm, vm,
                  pl.BlockSpec(memory_space=pl.ANY)],
        out_specs=vm,
        out_shape=jax.ShapeDtypeStruct((1, _D), jnp.float32),
        scratch_shapes=[
            pltpu.VMEM((_BLK, _D), jnp.float32),
            pltpu.VMEM((_BLK, _D), jnp.float32),
            pltpu.VMEM((_BLK, _D), jnp.float32),
            pltpu.VMEM((_BLK, _D), jnp.float32),
            pltpu.VMEM((_N // 128, 128), jnp.float32),
            pltpu.VMEM((_K, _D), jnp.float32),
            pltpu.SemaphoreType.DMA,
            pltpu.SemaphoreType.DMA,
            pltpu.SemaphoreType.DMA,
            pltpu.SemaphoreType.DMA,
            pltpu.SemaphoreType.DMA,
        ],
    )(c2, Wq, bq.reshape(1, _H), Wk, Wv, bv.reshape(1, _H), Wo,
      bo.reshape(1, _D), entity_embeddings)

    return out.reshape(_D)
